# baseline (speedup 1.0000x reference)
import functools

import jax
import jax.numpy as jnp
from jax import lax
from jax.experimental import pallas as pl
from jax.experimental.pallas import tpu as pltpu
from jax.experimental.pallas import tpu_sc as plsc

N_NODES = 10000
N_EDGES = 320000
D_FEAT = 128
D_HID = 64
D_OUT = 40
D_OUT_PAD = 48

NC, NS = 2, 16
NW = NC * NS
EPW = N_EDGES // NW
BLK = 128
NB = 79
EPW_PAD = NB * BLK
GBLK = 512
NBG = EPW_PAD // GBLK
SSUB = GBLK // BLK
CHUNKS = EPW // 16
DEG_P = EPW_PAD
ACC_ROWS = 10240
RPT = ACC_ROWS // NS
DUMMY = N_NODES
RB = 1000


def _sc_mesh():
    return plsc.VectorSubcoreMesh(
        core_axis_name="c", subcore_axis_name="s", num_cores=NC, num_subcores=NS
    )


@functools.partial(
    pl.kernel,
    out_type=jax.ShapeDtypeStruct((NW, 2 * DEG_P), jnp.float32),
    mesh=_sc_mesh(),
    scratch_types=[
        pltpu.VMEM((EPW,), jnp.int32),
        pltpu.VMEM((EPW,), jnp.int32),
        pltpu.VMEM((2 * DEG_P,), jnp.float32),
    ],
    compiler_params=pltpu.CompilerParams(
        needs_layout_passes=False, use_tc_tiling_on_sc=False
    ),
)
def _deg_kernel(data, out, srcv, dstv, acc):
    c = lax.axis_index("c")
    s = lax.axis_index("s")
    wid = c * NS + s
    pltpu.sync_copy(data.at[0, pl.ds(wid * EPW, EPW)], srcv)
    pltpu.sync_copy(data.at[1, pl.ds(wid * EPW, EPW)], dstv)
    zeros = jnp.zeros((16,), jnp.float32)

    def zbody(i, carry):
        acc[pl.ds(i * 16, 16)] = zeros
        return carry

    lax.fori_loop(0, 2 * DEG_P // 16, zbody, None)

    ones = jnp.ones((16,), jnp.float32)

    def hbody(i, carry):
        plsc.addupdate_scatter(acc, [srcv[pl.ds(i * 16, 16)]], ones)
        plsc.addupdate_scatter(acc, [dstv[pl.ds(i * 16, 16)] + DEG_P], ones)
        return carry

    lax.fori_loop(0, CHUNKS, hbody, None)
    pltpu.sync_copy(acc, out.at[wid])


def _make_agg(d):
    @functools.partial(
        pl.kernel,
        out_type=jax.ShapeDtypeStruct((NC, ACC_ROWS, d), jnp.float32),
        mesh=_sc_mesh(),
        scratch_types=[
            pltpu.VMEM((NB, BLK), jnp.int32),
            pltpu.VMEM((NB, BLK), jnp.int32),
            pltpu.VMEM((2, BLK, d), jnp.float32),
            pltpu.VMEM_SHARED((ACC_ROWS, d), jnp.float32),
            pltpu.SemaphoreType.DMA,
            pltpu.SemaphoreType.DMA,
        ],
        compiler_params=pltpu.CompilerParams(
            needs_layout_passes=False, use_tc_tiling_on_sc=False
        ),
    )
    def agg(h, srcp, dstp, out, srcv, dstv, gbuf, acc, sem0, sem1):
        c = lax.axis_index("c")
        s = lax.axis_index("s")
        wid = c * NS + s
        pltpu.sync_copy(srcp.at[wid], srcv)
        pltpu.sync_copy(dstp.at[wid], dstv)

        zeros = jnp.zeros((16,), jnp.float32)

        def zbody(r, carry):
            for k in range(d // 16):
                gbuf[0, r, pl.ds(k * 16, 16)] = zeros
            return carry

        lax.fori_loop(0, BLK, zbody, None)
        base = s * RPT
        for k in range(RPT // BLK):
            pltpu.sync_copy(gbuf.at[0], acc.at[pl.ds(base + k * BLK, BLK)])
        plsc.subcore_barrier()

        pltpu.async_copy(h.at[srcv.at[0]], gbuf.at[0], sem0)

        def step(i, carry):
            j0 = 2 * i
            j1 = 2 * i + 1
            j2 = 2 * i + 2

            @pl.when(j1 < NB)
            def _start1():
                pltpu.async_copy(h.at[srcv.at[j1]], gbuf.at[1], sem1)

            pltpu.make_async_copy(h.at[srcv.at[j0]], gbuf.at[0], sem0).wait()
            pltpu.sync_copy(gbuf.at[0], acc.at[dstv.at[j0]], add=True)

            @pl.when(j2 < NB)
            def _start2():
                pltpu.async_copy(h.at[srcv.at[j2]], gbuf.at[0], sem0)

            @pl.when(j1 < NB)
            def _drain1():
                pltpu.make_async_copy(h.at[srcv.at[j1]], gbuf.at[1], sem1).wait()
                pltpu.sync_copy(gbuf.at[1], acc.at[dstv.at[j1]], add=True)

            return carry

        lax.fori_loop(0, (NB + 1) // 2, step, None)
        plsc.subcore_barrier()
        pltpu.sync_copy(acc.at[pl.ds(base, RPT)], out.at[c, pl.ds(base, RPT)])

    return agg


_agg_hid = _make_agg(D_HID)
_agg_out = _make_agg(D_OUT_PAD)


def _matmul(x, w):
    n, kdim = x.shape
    dout = w.shape[1]

    def body(x_ref, w_ref, out_ref):
        out_ref[...] = jnp.dot(
            x_ref[...],
            w_ref[...],
            precision=lax.Precision.HIGHEST,
            preferred_element_type=jnp.float32,
        )

    return pl.pallas_call(
        body,
        grid=(n // RB,),
        out_shape=jax.ShapeDtypeStruct((n, dout), jnp.float32),
        in_specs=[
            pl.BlockSpec((RB, kdim), lambda i: (i, 0)),
            pl.BlockSpec((kdim, dout), lambda i: (0, 0)),
        ],
        out_specs=pl.BlockSpec((RB, dout), lambda i: (i, 0)),
    )(x, w)


def _norm_scale(deg_part, h1):
    def body(dp_ref, h_ref, nm_ref, out_ref):
        deg = jnp.sum(dp_ref[...], axis=0)
        nmt = jnp.transpose(lax.rsqrt(jnp.maximum(deg, 1.0)))
        nm_ref[...] = nmt
        out_ref[...] = h_ref[...] * nmt[:N_NODES, 0:1]

    return pl.pallas_call(
        body,
        out_shape=(
            jax.ShapeDtypeStruct((DEG_P, 2), jnp.float32),
            jax.ShapeDtypeStruct((N_NODES, D_HID), jnp.float32),
        ),
        in_specs=[
            pl.BlockSpec((NW, 2, DEG_P), lambda: (0, 0, 0)),
            pl.BlockSpec((N_NODES, D_HID), lambda: (0, 0)),
        ],
        out_specs=(
            pl.BlockSpec((DEG_P, 2), lambda: (0, 0)),
            pl.BlockSpec((N_NODES, D_HID), lambda: (0, 0)),
        ),
    )(deg_part, h1)


def _mid_layer(agg, nm, b1, w2p):
    def body(a_ref, nm_ref, b_ref, w_ref, out_ref):
        t = a_ref[0] + a_ref[1]
        t = jnp.maximum(t * nm_ref[:, 1:2] + b_ref[...], 0.0)
        out_ref[...] = jnp.dot(
            t * nm_ref[:, 0:1],
            w_ref[...],
            precision=lax.Precision.HIGHEST,
            preferred_element_type=jnp.float32,
        )

    return pl.pallas_call(
        body,
        grid=(N_NODES // RB,),
        out_shape=jax.ShapeDtypeStruct((N_NODES, D_OUT_PAD), jnp.float32),
        in_specs=[
            pl.BlockSpec((NC, RB, D_HID), lambda i: (0, i, 0)),
            pl.BlockSpec((RB, 2), lambda i: (i, 0)),
            pl.BlockSpec((1, D_HID), lambda i: (0, 0)),
            pl.BlockSpec((D_HID, D_OUT_PAD), lambda i: (0, 0)),
        ],
        out_specs=pl.BlockSpec((RB, D_OUT_PAD), lambda i: (i, 0)),
    )(agg, nm, b1, w2p)


def _final_layer(agg, nm, b2):
    def body(a_ref, nm_ref, b_ref, out_ref):
        t = a_ref[0] + a_ref[1]
        out_ref[...] = t[:, :D_OUT] * nm_ref[:, 1:2] + b_ref[...]

    return pl.pallas_call(
        body,
        grid=(N_NODES // RB,),
        out_shape=jax.ShapeDtypeStruct((N_NODES, D_OUT), jnp.float32),
        in_specs=[
            pl.BlockSpec((NC, RB, D_OUT_PAD), lambda i: (0, i, 0)),
            pl.BlockSpec((RB, 2), lambda i: (i, 0)),
            pl.BlockSpec((1, D_OUT), lambda i: (0, 0)),
        ],
        out_specs=pl.BlockSpec((RB, D_OUT), lambda i: (i, 0)),
    )(agg, nm, b2)


def kernel(data, feats, W1, b1, W2, b2):
    data32 = data.astype(jnp.int32)
    src = data32[0]
    dst = data32[1]
    srcp = jnp.pad(
        src.reshape(NW, EPW), ((0, 0), (0, EPW_PAD - EPW))
    ).reshape(NW, NB, BLK)
    dstp = jnp.concatenate(
        [
            dst.reshape(NW, EPW),
            jnp.broadcast_to(
                DUMMY + jnp.arange(NW, dtype=jnp.int32)[:, None],
                (NW, EPW_PAD - EPW),
            ),
        ],
        axis=1,
    ).reshape(NW, NB, BLK)

    deg_part = _deg_kernel(data32).reshape(NW, 2, DEG_P)
    h1 = _matmul(feats, W1)
    nm, h1s = _norm_scale(deg_part, h1)
    agg1 = _agg_hid(h1s, srcp, dstp)
    w2p = jnp.pad(W2, ((0, 0), (0, D_OUT_PAD - D_OUT)))
    h2 = _mid_layer(agg1, nm, b1.reshape(1, -1), w2p)
    agg2 = _agg_out(h2, srcp, dstp)
    return _final_layer(agg2, nm, b2.reshape(1, -1))


# speedup vs baseline: 1.0056x; 1.0056x over previous
import functools

import jax
import jax.numpy as jnp
from jax import lax
from jax.experimental import pallas as pl
from jax.experimental.pallas import tpu as pltpu
from jax.experimental.pallas import tpu_sc as plsc

N_NODES = 10000
N_EDGES = 320000
D_FEAT = 128
D_HID = 64
D_OUT = 40
D_OUT_PAD = 48

NC, NS = 2, 16
NW = NC * NS
EPW = N_EDGES // NW
BLK = 128
NB = 79
EPW_PAD = NB * BLK
GBLK = 512
NBG = EPW_PAD // GBLK
SSUB = GBLK // BLK
CHUNKS = EPW // 16
DEG_P = EPW_PAD
ACC_ROWS = 10240
RPT = ACC_ROWS // NS
DUMMY = N_NODES
RB = 1000


def _sc_mesh():
    return plsc.VectorSubcoreMesh(
        core_axis_name="c", subcore_axis_name="s", num_cores=NC, num_subcores=NS
    )


@functools.partial(
    pl.kernel,
    out_type=jax.ShapeDtypeStruct((NW, 2 * DEG_P), jnp.float32),
    mesh=_sc_mesh(),
    scratch_types=[
        pltpu.VMEM((NB, BLK), jnp.int32),
        pltpu.VMEM((NB, BLK), jnp.int32),
        pltpu.VMEM((2 * DEG_P,), jnp.float32),
    ],
    compiler_params=pltpu.CompilerParams(needs_layout_passes=False),
)
def _deg_kernel(srcp, dstp, out, srcv, dstv, acc):
    c = lax.axis_index("c")
    s = lax.axis_index("s")
    wid = c * NS + s
    pltpu.sync_copy(srcp.at[wid], srcv)
    pltpu.sync_copy(dstp.at[wid], dstv)
    zeros = jnp.zeros((16,), jnp.float32)

    def zbody(i, carry):
        acc[pl.ds(i * 16, 16)] = zeros
        return carry

    lax.fori_loop(0, 2 * DEG_P // 16, zbody, None)

    ones = jnp.ones((16,), jnp.float32)

    def hbody(i, carry):
        r = i >> 3
        col = (i & 7) * 16
        plsc.addupdate_scatter(acc, [srcv[r, pl.ds(col, 16)]], ones)
        plsc.addupdate_scatter(acc, [dstv[r, pl.ds(col, 16)] + DEG_P], ones)
        return carry

    lax.fori_loop(0, CHUNKS, hbody, None)
    pltpu.sync_copy(acc, out.at[wid])


def _make_agg(d):
    @functools.partial(
        pl.kernel,
        out_type=jax.ShapeDtypeStruct((NC, ACC_ROWS, d), jnp.float32),
        mesh=_sc_mesh(),
        scratch_types=[
            pltpu.VMEM((NB, BLK), jnp.int32),
            pltpu.VMEM((NB, BLK), jnp.int32),
            pltpu.VMEM((2, BLK, d), jnp.float32),
            pltpu.VMEM_SHARED((ACC_ROWS, d), jnp.float32),
            pltpu.SemaphoreType.DMA,
            pltpu.SemaphoreType.DMA,
        ],
        compiler_params=pltpu.CompilerParams(
            needs_layout_passes=False, use_tc_tiling_on_sc=False
        ),
    )
    def agg(h, srcp, dstp, out, srcv, dstv, gbuf, acc, sem0, sem1):
        c = lax.axis_index("c")
        s = lax.axis_index("s")
        wid = c * NS + s
        pltpu.sync_copy(srcp.at[wid], srcv)
        pltpu.sync_copy(dstp.at[wid], dstv)

        zeros = jnp.zeros((16,), jnp.float32)

        def zbody(r, carry):
            for k in range(d // 16):
                gbuf[0, r, pl.ds(k * 16, 16)] = zeros
            return carry

        lax.fori_loop(0, BLK, zbody, None)
        base = s * RPT
        for k in range(RPT // BLK):
            pltpu.sync_copy(gbuf.at[0], acc.at[pl.ds(base + k * BLK, BLK)])
        plsc.subcore_barrier()

        pltpu.async_copy(h.at[srcv.at[0]], gbuf.at[0], sem0)

        def step(i, carry):
            j0 = 2 * i
            j1 = 2 * i + 1
            j2 = 2 * i + 2

            @pl.when(j1 < NB)
            def _start1():
                pltpu.async_copy(h.at[srcv.at[j1]], gbuf.at[1], sem1)

            pltpu.make_async_copy(h.at[srcv.at[j0]], gbuf.at[0], sem0).wait()
            pltpu.sync_copy(gbuf.at[0], acc.at[dstv.at[j0]], add=True)

            @pl.when(j2 < NB)
            def _start2():
                pltpu.async_copy(h.at[srcv.at[j2]], gbuf.at[0], sem0)

            @pl.when(j1 < NB)
            def _drain1():
                pltpu.make_async_copy(h.at[srcv.at[j1]], gbuf.at[1], sem1).wait()
                pltpu.sync_copy(gbuf.at[1], acc.at[dstv.at[j1]], add=True)

            return carry

        lax.fori_loop(0, (NB + 1) // 2, step, None)
        plsc.subcore_barrier()
        pltpu.sync_copy(acc.at[pl.ds(base, RPT)], out.at[c, pl.ds(base, RPT)])

    return agg


_agg_hid = _make_agg(D_HID)
_agg_out = _make_agg(D_OUT_PAD)


def _matmul(x, w):
    n, kdim = x.shape
    dout = w.shape[1]

    def body(x_ref, w_ref, out_ref):
        out_ref[...] = jnp.dot(
            x_ref[...],
            w_ref[...],
            precision=lax.Precision.HIGHEST,
            preferred_element_type=jnp.float32,
        )

    return pl.pallas_call(
        body,
        grid=(n // RB,),
        out_shape=jax.ShapeDtypeStruct((n, dout), jnp.float32),
        in_specs=[
            pl.BlockSpec((RB, kdim), lambda i: (i, 0)),
            pl.BlockSpec((kdim, dout), lambda i: (0, 0)),
        ],
        out_specs=pl.BlockSpec((RB, dout), lambda i: (i, 0)),
    )(x, w)


def _norm_scale(deg_part, h1):
    def body(dp_ref, h_ref, nm_ref, out_ref):
        deg = jnp.sum(dp_ref[...], axis=0)
        nmt = jnp.transpose(lax.rsqrt(jnp.maximum(deg, 1.0)))
        nm_ref[...] = nmt
        out_ref[...] = h_ref[...] * nmt[:N_NODES, 0:1]

    return pl.pallas_call(
        body,
        out_shape=(
            jax.ShapeDtypeStruct((DEG_P, 2), jnp.float32),
            jax.ShapeDtypeStruct((N_NODES, D_HID), jnp.float32),
        ),
        in_specs=[
            pl.BlockSpec((NW, 2, DEG_P), lambda: (0, 0, 0)),
            pl.BlockSpec((N_NODES, D_HID), lambda: (0, 0)),
        ],
        out_specs=(
            pl.BlockSpec((DEG_P, 2), lambda: (0, 0)),
            pl.BlockSpec((N_NODES, D_HID), lambda: (0, 0)),
        ),
    )(deg_part, h1)


def _mid_layer(agg, nm, b1, w2p):
    def body(a_ref, nm_ref, b_ref, w_ref, out_ref):
        t = a_ref[0] + a_ref[1]
        t = jnp.maximum(t * nm_ref[:, 1:2] + b_ref[...], 0.0)
        out_ref[...] = jnp.dot(
            t * nm_ref[:, 0:1],
            w_ref[...],
            precision=lax.Precision.HIGHEST,
            preferred_element_type=jnp.float32,
        )

    return pl.pallas_call(
        body,
        grid=(N_NODES // RB,),
        out_shape=jax.ShapeDtypeStruct((N_NODES, D_OUT_PAD), jnp.float32),
        in_specs=[
            pl.BlockSpec((NC, RB, D_HID), lambda i: (0, i, 0)),
            pl.BlockSpec((RB, 2), lambda i: (i, 0)),
            pl.BlockSpec((1, D_HID), lambda i: (0, 0)),
            pl.BlockSpec((D_HID, D_OUT_PAD), lambda i: (0, 0)),
        ],
        out_specs=pl.BlockSpec((RB, D_OUT_PAD), lambda i: (i, 0)),
    )(agg, nm, b1, w2p)


def _final_layer(agg, nm, b2):
    def body(a_ref, nm_ref, b_ref, out_ref):
        t = a_ref[0] + a_ref[1]
        out_ref[...] = t[:, :D_OUT] * nm_ref[:, 1:2] + b_ref[...]

    return pl.pallas_call(
        body,
        grid=(N_NODES // RB,),
        out_shape=jax.ShapeDtypeStruct((N_NODES, D_OUT), jnp.float32),
        in_specs=[
            pl.BlockSpec((NC, RB, D_OUT_PAD), lambda i: (0, i, 0)),
            pl.BlockSpec((RB, 2), lambda i: (i, 0)),
            pl.BlockSpec((1, D_OUT), lambda i: (0, 0)),
        ],
        out_specs=pl.BlockSpec((RB, D_OUT), lambda i: (i, 0)),
    )(agg, nm, b2)


def kernel(data, feats, W1, b1, W2, b2):
    data32 = data.astype(jnp.int32)
    src = data32[0]
    dst = data32[1]
    srcp = jnp.pad(
        src.reshape(NW, EPW), ((0, 0), (0, EPW_PAD - EPW))
    ).reshape(NW, NB, BLK)
    dstp = jnp.concatenate(
        [
            dst.reshape(NW, EPW),
            jnp.broadcast_to(
                DUMMY + jnp.arange(NW, dtype=jnp.int32)[:, None],
                (NW, EPW_PAD - EPW),
            ),
        ],
        axis=1,
    ).reshape(NW, NB, BLK)

    deg_part = _deg_kernel(srcp, dstp).reshape(NW, 2, DEG_P)
    h1 = _matmul(feats, W1)
    nm, h1s = _norm_scale(deg_part, h1)
    agg1 = _agg_hid(h1s, srcp, dstp)
    w2p = jnp.pad(W2, ((0, 0), (0, D_OUT_PAD - D_OUT)))
    h2 = _mid_layer(agg1, nm, b1.reshape(1, -1), w2p)
    agg2 = _agg_out(h2, srcp, dstp)
    return _final_layer(agg2, nm, b2.reshape(1, -1))
